# baseline (speedup 1.0000x reference)
import jax
import jax.numpy as jnp
from jax import lax
from jax.experimental import pallas as pl
from jax.experimental.pallas import tpu as pltpu
from jax.experimental.pallas import tpu_sc as plsc

B = 16384
D = 64
N = 100000
NA = 50048
NB = N - NA
CH = 4096
NC = 2
NW = 32


def _sc_gather3_t(users, pos_items, neg_items, ut_t, it_t):
    mesh = plsc.VectorSubcoreMesh(core_axis_name="core", subcore_axis_name="subcore")

    @pl.kernel(
        out_type=[
            jax.ShapeDtypeStruct((D, B), jnp.float32),
            jax.ShapeDtypeStruct((D, B), jnp.float32),
            jax.ShapeDtypeStruct((D, B), jnp.float32),
        ],
        mesh=mesh,
        compiler_params=pltpu.CompilerParams(needs_layout_passes=False),
        scratch_types=[
            pltpu.VMEM((1, NA), jnp.float32),
            pltpu.VMEM((1, NB), jnp.float32),
            pltpu.VMEM((B,), jnp.float32),
            pltpu.VMEM((CH,), jnp.int32),
            pltpu.VMEM((CH,), jnp.int32),
            pltpu.SemaphoreType.DMA,
            pltpu.SemaphoreType.DMA,
            pltpu.SemaphoreType.DMA,
            pltpu.SemaphoreType.DMA,
        ],
    )
    def k(u_hbm, p_hbm, n_hbm, ut_hbm, it_hbm, ou_hbm, op_hbm, on_hbm,
          rowa_v, rowb_v, outf_v, idx0_v, idx1_v,
          sem_rowa, sem_rowb, sem_idx, sem_out):
        wid = lax.axis_index("subcore") * NC + lax.axis_index("core")
        idx_bufs = (idx0_v, idx1_v)
        zeros16 = jnp.zeros((16,), jnp.int32)
        NCHUNK = B // CH

        def load_half(tbl_hbm, d, col0, total, dest_v, sem):
            sub = 12544
            hs = []
            lo = 0
            while lo < total:
                ln = min(sub, total - lo)
                hs.append(
                    pltpu.async_copy(
                        tbl_hbm.at[pl.ds(d, 1), pl.ds(col0 + lo, ln)],
                        dest_v.at[pl.ds(0, 1), pl.ds(lo, ln)],
                        sem,
                    )
                )
                lo += ln
            return hs

        d0, d1 = wid, wid + NW
        tasks = [
            (ut_hbm, d0, [(u_hbm, ou_hbm)]),
            (it_hbm, d0, [(p_hbm, op_hbm), (n_hbm, on_hbm)]),
            (ut_hbm, d1, [(u_hbm, ou_hbm)]),
            (it_hbm, d1, [(p_hbm, op_hbm), (n_hbm, on_hbm)]),
        ]
        sched = []
        for ti, (tbl, d, prs) in enumerate(tasks):
            for qi in range(len(prs)):
                for half in range(2):
                    for c in range(NCHUNK):
                        sched.append((ti, qi, half, c))

        idx_pending = [None] * len(sched)
        out_pending = [None] * NCHUNK

        def fire_idx(g):
            ti, qi, half, c = sched[g]
            idx_hbm = tasks[ti][2][qi][0]
            idx_pending[g] = pltpu.async_copy(
                idx_hbm.at[pl.ds(c * CH, CH)], idx_bufs[g % 2], sem_idx
            )

        rowa_handles = load_half(tasks[0][0], tasks[0][1], 0, NA, rowa_v, sem_rowa)
        rowb_handles = None
        fire_idx(0)

        for g, (ti, qi, half, c) in enumerate(sched):
            tbl_hbm, d, prs = tasks[ti]
            idx_hbm, out_hbm = prs[qi]
            first_of_task = qi == 0 and half == 0 and c == 0
            if first_of_task:
                rowb_handles = load_half(tbl_hbm, d, NA, NB, rowb_v, sem_rowb)
                for h in rowa_handles:
                    h.wait()
            if qi == 0 and half == 1 and c == 0:
                for h in rowb_handles:
                    h.wait()
            if g + 1 < len(sched):
                fire_idx(g + 1)
            idx_pending[g].wait()
            ib = idx_bufs[g % 2]
            base = c * CH

            if half == 0:
                if out_pending[c] is not None:
                    out_pending[c].wait()

                @pl.loop(0, CH, step=128)
                def _(j):
                    ivs = [ib[pl.ds(j + 16 * t, 16)] for t in range(8)]
                    ms = [iv < NA for iv in ivs]
                    gs = [
                        plsc.load_gather(rowa_v, [zeros16, iv], mask=m)
                        for iv, m in zip(ivs, ms)
                    ]
                    for t in range(8):
                        outf_v[pl.ds(base + j + 16 * t, 16)] = gs[t]
            else:

                @pl.loop(0, CH, step=128)
                def _(j):
                    ivs = [ib[pl.ds(j + 16 * t, 16)] for t in range(8)]
                    mbs = [iv >= NA for iv in ivs]
                    ivbs = [
                        jnp.where(mb, iv - NA, 0) for iv, mb in zip(ivs, mbs)
                    ]
                    gs = [
                        plsc.load_gather(rowb_v, [zeros16, ivb], mask=mb)
                        for ivb, mb in zip(ivbs, mbs)
                    ]
                    for t in range(8):
                        sl = pl.ds(base + j + 16 * t, 16)
                        outf_v[sl] = jnp.where(mbs[t], gs[t], outf_v[sl])

                out_pending[c] = pltpu.async_copy(
                    outf_v.at[pl.ds(base, CH)],
                    out_hbm.at[d, pl.ds(base, CH)],
                    sem_out,
                )
            is_last_a = qi == len(prs) - 1 and half == 0 and c == NCHUNK - 1
            if is_last_a and ti + 1 < len(tasks):
                nt = tasks[ti + 1]
                rowa_handles = load_half(nt[0], nt[1], 0, NA, rowa_v, sem_rowa)

        for h in out_pending:
            if h is not None:
                h.wait()

    return k(users, pos_items, neg_items, ut_t, it_t)


def _tc_norm_t(u_t, p_t, n_t):
    blk = 2048

    def body(u_ref, p_ref, n_ref, o_ref):
        uu = u_ref[...]
        pp = p_ref[...]
        nn = n_ref[...]
        o_ref[...] = (
            jnp.sum(uu * uu, axis=0)
            + jnp.sum(pp * pp, axis=0)
            + jnp.sum(nn * nn, axis=0)
        )

    return pl.pallas_call(
        body,
        grid=(B // blk,),
        in_specs=[
            pl.BlockSpec((D, blk), lambda i: (0, i)),
            pl.BlockSpec((D, blk), lambda i: (0, i)),
            pl.BlockSpec((D, blk), lambda i: (0, i)),
        ],
        out_specs=pl.BlockSpec((blk,), lambda i: (i,)),
        out_shape=jax.ShapeDtypeStruct((B,), jnp.float32),
    )(u_t, p_t, n_t)


def kernel(users, pos_items, neg_items, user_table, item_table):
    users = users.astype(jnp.int32)
    pos_items = pos_items.astype(jnp.int32)
    neg_items = neg_items.astype(jnp.int32)
    ut_t = user_table.T
    it_t = item_table.T
    ou_t, op_t, on_t = _sc_gather3_t(users, pos_items, neg_items, ut_t, it_t)
    l2 = _tc_norm_t(ou_t, op_t, on_t)
    return (ou_t.T, op_t.T, on_t.T, l2)


# speedup vs baseline: 1.2193x; 1.2193x over previous
import jax
import jax.numpy as jnp
from jax import lax
from jax.experimental import pallas as pl
from jax.experimental.pallas import tpu as pltpu
from jax.experimental.pallas import tpu_sc as plsc

B = 16384
D = 64
N = 100000
CH = 4096
NC = 2
NW = 32


def _sc_gather3_t(users, pos_items, neg_items, ut_t, it_t):
    mesh = plsc.VectorSubcoreMesh(core_axis_name="core", subcore_axis_name="subcore")

    @pl.kernel(
        out_type=[
            jax.ShapeDtypeStruct((D, B), jnp.float32),
            jax.ShapeDtypeStruct((D, B), jnp.float32),
            jax.ShapeDtypeStruct((D, B), jnp.float32),
        ],
        mesh=mesh,
        compiler_params=pltpu.CompilerParams(needs_layout_passes=False),
        scratch_types=[
            pltpu.VMEM((1, N), jnp.float32),
            pltpu.VMEM((CH,), jnp.int32),
            pltpu.VMEM((CH,), jnp.int32),
            pltpu.VMEM((CH,), jnp.float32),
            pltpu.VMEM((CH,), jnp.float32),
            pltpu.SemaphoreType.DMA,
            pltpu.SemaphoreType.DMA,
            pltpu.SemaphoreType.DMA,
        ],
    )
    def k(u_hbm, p_hbm, n_hbm, ut_hbm, it_hbm, ou_hbm, op_hbm, on_hbm,
          row_v, idx0_v, idx1_v, out0_v, out1_v, sem_row, sem_idx, sem_out):
        wid = lax.axis_index("subcore") * NC + lax.axis_index("core")
        idx_bufs = (idx0_v, idx1_v)
        out_bufs = (out0_v, out1_v)
        zeros16 = jnp.zeros((16,), jnp.int32)
        NCHUNK = B // CH

        def load_row(tbl_hbm, d):
            nsub = 8
            sub = 12544
            hs = []
            for s in range(nsub):
                lo = s * sub
                ln = min(sub, N - lo)
                hs.append(
                    pltpu.async_copy(
                        tbl_hbm.at[pl.ds(d, 1), pl.ds(lo, ln)],
                        row_v.at[pl.ds(0, 1), pl.ds(lo, ln)],
                        sem_row,
                    )
                )
            return hs

        d0, d1 = wid, wid + NW
        passes = [
            (ut_hbm, d0, u_hbm, ou_hbm, 0),
            (it_hbm, d0, p_hbm, op_hbm, 1),
            (it_hbm, d0, n_hbm, on_hbm, 1),
            (ut_hbm, d1, u_hbm, ou_hbm, 2),
            (it_hbm, d1, p_hbm, op_hbm, 3),
            (it_hbm, d1, n_hbm, on_hbm, 3),
        ]
        sched = [(pi, c) for pi in range(len(passes)) for c in range(NCHUNK)]

        idx_pending = [None] * len(sched)
        out_pending = [None, None]

        def fire_idx(g):
            pi, c = sched[g]
            idx_hbm = passes[pi][2]
            idx_pending[g] = pltpu.async_copy(
                idx_hbm.at[pl.ds(c * CH, CH)], idx_bufs[g % 2], sem_idx
            )

        row_handles = load_row(passes[0][0], passes[0][1])
        fire_idx(0)

        for g, (pi, c) in enumerate(sched):
            tbl_hbm, d, idx_hbm, out_hbm, tag = passes[pi]
            if g == 0 or (c == 0 and tag != passes[pi - 1][4]):
                for h in row_handles:
                    h.wait()
            if g + 1 < len(sched):
                fire_idx(g + 1)
            idx_pending[g].wait()
            if out_pending[g % 2] is not None:
                out_pending[g % 2].wait()
            ib = idx_bufs[g % 2]
            ob = out_bufs[g % 2]

            @pl.loop(0, CH, step=256)
            def _(j):
                ivs = [ib[pl.ds(j + 16 * t, 16)] for t in range(16)]
                gs = [plsc.load_gather(row_v, [zeros16, iv]) for iv in ivs]
                for t in range(16):
                    ob[pl.ds(j + 16 * t, 16)] = gs[t]

            out_pending[g % 2] = pltpu.async_copy(
                ob, out_hbm.at[d, pl.ds(c * CH, CH)], sem_out
            )
            if g + 1 < len(sched):
                npi, nc = sched[g + 1]
                if nc == 0 and passes[npi][4] != tag:
                    row_handles = load_row(passes[npi][0], passes[npi][1])

        for h in out_pending:
            if h is not None:
                h.wait()

    return k(users, pos_items, neg_items, ut_t, it_t)


def _tc_norm_t(u_t, p_t, n_t):
    blk = 2048

    def body(u_ref, p_ref, n_ref, o_ref):
        uu = u_ref[...]
        pp = p_ref[...]
        nn = n_ref[...]
        o_ref[...] = (
            jnp.sum(uu * uu, axis=0)
            + jnp.sum(pp * pp, axis=0)
            + jnp.sum(nn * nn, axis=0)
        )

    return pl.pallas_call(
        body,
        grid=(B // blk,),
        in_specs=[
            pl.BlockSpec((D, blk), lambda i: (0, i)),
            pl.BlockSpec((D, blk), lambda i: (0, i)),
            pl.BlockSpec((D, blk), lambda i: (0, i)),
        ],
        out_specs=pl.BlockSpec((blk,), lambda i: (i,)),
        out_shape=jax.ShapeDtypeStruct((B,), jnp.float32),
    )(u_t, p_t, n_t)


def kernel(users, pos_items, neg_items, user_table, item_table):
    users = users.astype(jnp.int32)
    pos_items = pos_items.astype(jnp.int32)
    neg_items = neg_items.astype(jnp.int32)
    ut_t = user_table.T
    it_t = item_table.T
    ou_t, op_t, on_t = _sc_gather3_t(users, pos_items, neg_items, ut_t, it_t)
    l2 = _tc_norm_t(ou_t, op_t, on_t)
    return (ou_t.T, op_t.T, on_t.T, l2)


# speedup vs baseline: 1.2402x; 1.0171x over previous
import jax
import jax.numpy as jnp
from jax import lax
from jax.experimental import pallas as pl
from jax.experimental.pallas import tpu as pltpu
from jax.experimental.pallas import tpu_sc as plsc

B = 16384
D = 64
N = 100000
CH = 4096
NC = 2
NW = 32


def _sc_gather3_t(users, pos_items, neg_items, ut_t, it_t):
    mesh = plsc.VectorSubcoreMesh(core_axis_name="core", subcore_axis_name="subcore")

    @pl.kernel(
        out_type=[
            jax.ShapeDtypeStruct((D, B), jnp.float32),
            jax.ShapeDtypeStruct((D, B), jnp.float32),
            jax.ShapeDtypeStruct((D, B), jnp.float32),
        ],
        mesh=mesh,
        compiler_params=pltpu.CompilerParams(needs_layout_passes=False),
        scratch_types=[
            pltpu.VMEM((1, N), jnp.float32),
            pltpu.VMEM((CH,), jnp.int32),
            pltpu.VMEM((CH,), jnp.int32),
            pltpu.VMEM((CH,), jnp.float32),
            pltpu.VMEM((CH,), jnp.float32),
            pltpu.SemaphoreType.DMA,
            pltpu.SemaphoreType.DMA,
            pltpu.SemaphoreType.DMA,
        ],
    )
    def k(u_hbm, p_hbm, n_hbm, ut_hbm, it_hbm, ou_hbm, op_hbm, on_hbm,
          row_v, idx0_v, idx1_v, out0_v, out1_v, sem_row, sem_idx, sem_out):
        wid = lax.axis_index("subcore") * NC + lax.axis_index("core")
        idx_bufs = (idx0_v, idx1_v)
        out_bufs = (out0_v, out1_v)
        zeros16 = jnp.zeros((16,), jnp.int32)
        NCHUNK = B // CH

        def load_row(tbl_hbm, d):
            nsub = 8
            sub = 12544
            hs = []
            for s in range(nsub):
                lo = s * sub
                ln = min(sub, N - lo)
                hs.append(
                    pltpu.async_copy(
                        tbl_hbm.at[pl.ds(d, 1), pl.ds(lo, ln)],
                        row_v.at[pl.ds(0, 1), pl.ds(lo, ln)],
                        sem_row,
                    )
                )
            return hs

        d0, d1 = wid, wid + NW
        passes = [
            (ut_hbm, d0, u_hbm, ou_hbm, 0),
            (it_hbm, d0, p_hbm, op_hbm, 1),
            (it_hbm, d0, n_hbm, on_hbm, 1),
            (ut_hbm, d1, u_hbm, ou_hbm, 2),
            (it_hbm, d1, p_hbm, op_hbm, 3),
            (it_hbm, d1, n_hbm, on_hbm, 3),
        ]
        sched = [(pi, c) for pi in range(len(passes)) for c in range(NCHUNK)]

        idx_pending = [None] * len(sched)
        out_pending = [None, None]

        def fire_idx(g):
            pi, c = sched[g]
            idx_hbm = passes[pi][2]
            idx_pending[g] = pltpu.async_copy(
                idx_hbm.at[pl.ds(c * CH, CH)], idx_bufs[g % 2], sem_idx
            )

        row_handles = load_row(passes[0][0], passes[0][1])
        fire_idx(0)

        for g, (pi, c) in enumerate(sched):
            tbl_hbm, d, idx_hbm, out_hbm, tag = passes[pi]
            if g == 0 or (c == 0 and tag != passes[pi - 1][4]):
                for h in row_handles:
                    h.wait()
            if g + 1 < len(sched):
                fire_idx(g + 1)
            idx_pending[g].wait()
            if out_pending[g % 2] is not None:
                out_pending[g % 2].wait()
            ib = idx_bufs[g % 2]
            ob = out_bufs[g % 2]

            @pl.loop(0, CH, step=128)
            def _(j):
                ivs = [ib[pl.ds(j + 16 * t, 16)] for t in range(8)]
                gs = [plsc.load_gather(row_v, [zeros16, iv]) for iv in ivs]
                for t in range(8):
                    ob[pl.ds(j + 16 * t, 16)] = gs[t]

            out_pending[g % 2] = pltpu.async_copy(
                ob, out_hbm.at[d, pl.ds(c * CH, CH)], sem_out
            )
            if g + 1 < len(sched):
                npi, nc = sched[g + 1]
                if nc == 0 and passes[npi][4] != tag:
                    row_handles = load_row(passes[npi][0], passes[npi][1])

        for h in out_pending:
            if h is not None:
                h.wait()

    return k(users, pos_items, neg_items, ut_t, it_t)


def _tc_norm_t(u_t, p_t, n_t):
    blk = 2048

    def body(u_ref, p_ref, n_ref, o_ref):
        uu = u_ref[...]
        pp = p_ref[...]
        nn = n_ref[...]
        o_ref[...] = (
            jnp.sum(uu * uu, axis=0)
            + jnp.sum(pp * pp, axis=0)
            + jnp.sum(nn * nn, axis=0)
        )

    return pl.pallas_call(
        body,
        grid=(B // blk,),
        in_specs=[
            pl.BlockSpec((D, blk), lambda i: (0, i)),
            pl.BlockSpec((D, blk), lambda i: (0, i)),
            pl.BlockSpec((D, blk), lambda i: (0, i)),
        ],
        out_specs=pl.BlockSpec((blk,), lambda i: (i,)),
        out_shape=jax.ShapeDtypeStruct((B,), jnp.float32),
    )(u_t, p_t, n_t)


def kernel(users, pos_items, neg_items, user_table, item_table):
    users = users.astype(jnp.int32)
    pos_items = pos_items.astype(jnp.int32)
    neg_items = neg_items.astype(jnp.int32)
    ut_t = user_table.T
    it_t = item_table.T
    ou_t, op_t, on_t = _sc_gather3_t(users, pos_items, neg_items, ut_t, it_t)
    l2 = _tc_norm_t(ou_t, op_t, on_t)
    return (ou_t.T, op_t.T, on_t.T, l2)
